# baseline (speedup 1.0000x reference)
import functools

import jax
import jax.numpy as jnp
from jax import lax
from jax.experimental import pallas as pl
from jax.experimental.pallas import tpu as pltpu
from jax.experimental.pallas import tpu_sc as plsc

NC = 2
NS = 16
NW = NC * NS
LANES = 16
LB = 2


@functools.lru_cache(maxsize=None)
def _build(B, L, V, E):
    assert E % NW == 0 and L % LB == 0 and B % LANES == 0
    e_per_w = E // NW
    n_blk = L // LB

    mesh = plsc.VectorSubcoreMesh(
        core_axis_name="c", subcore_axis_name="s", num_cores=NC, num_subcores=NS
    )

    def body(xt_hbm, tokt_hbm, posf_hbm, w_hbm,
             col_v, pos_v, xb0, xb1, ob0, ob1, cs, xs0, xs1, os0, os1):
        xbufs = (xb0, xb1)
        obufs = (ob0, ob1)
        xsems = (xs0, xs1)
        osems = (os0, os1)
        wid = lax.axis_index("s") * NC + lax.axis_index("c")
        e0 = wid * e_per_w

        pltpu.sync_copy(posf_hbm, pos_v)

        def xload(blk, b):
            return pltpu.make_async_copy(
                xt_hbm.at[pl.ds(blk * LB, LB)], xbufs[b], xsems[b]
            )

        def oscatter(blk, e, b):
            return pltpu.make_async_copy(
                obufs[b], w_hbm.at[pl.ds(blk * LB, LB), e], osems[b]
            )

        def compute_blk(blk, e, b):
            xb, ob = xbufs[b], obufs[b]
            pos_base = e * L
            for lb in range(LB):
                l = blk * LB + lb
                pos = plsc.load_gather(
                    pos_v, [jnp.full((LANES,), pos_base + l, jnp.int32)]
                )

                def per_k(k, _, xb=xb, ob=ob, lb=lb, pos=pos):
                    c = k * LANES
                    v = xb[lb, pl.ds(c, LANES)]
                    val = plsc.load_gather(col_v, [v])
                    ob[lb, pl.ds(c, LANES)] = val + pos
                    return 0

                lax.fori_loop(0, B // LANES, per_k, 0)

        for p in range(e_per_w):
            e = e0 + p
            pltpu.async_copy(tokt_hbm.at[e], col_v, cs).wait()

            xload(0, 0).start()
            xload(1, 1).start()

            def pair_body(t, _, e=e):
                for b in range(2):
                    blk = 2 * t + b
                    xload(blk, b).wait()

                    @pl.when(t > 0)
                    def _():
                        oscatter(blk - 2, e, b).wait()

                    compute_blk(blk, e, b)

                    @pl.when(blk + 2 < n_blk)
                    def _():
                        xload(blk + 2, b).start()

                    oscatter(blk, e, b).start()
                return 0

            lax.fori_loop(0, n_blk // 2, pair_body, 0)

            oscatter(n_blk - 2, e, 0).wait()
            oscatter(n_blk - 1, e, 1).wait()

    return pl.kernel(
        body,
        out_type=jax.ShapeDtypeStruct((L, E, B), jnp.float32),
        mesh=mesh,
        compiler_params=pltpu.CompilerParams(
            use_tc_tiling_on_sc=False, needs_layout_passes=False
        ),
        scratch_types=[
            pltpu.VMEM((V,), jnp.float32),
            pltpu.VMEM((E * L,), jnp.float32),
            pltpu.VMEM((LB, B), jnp.int32),
            pltpu.VMEM((LB, B), jnp.int32),
            pltpu.VMEM((LB, B), jnp.float32),
            pltpu.VMEM((LB, B), jnp.float32),
            pltpu.SemaphoreType.DMA,
            pltpu.SemaphoreType.DMA,
            pltpu.SemaphoreType.DMA,
            pltpu.SemaphoreType.DMA,
            pltpu.SemaphoreType.DMA,
        ],
    )


def kernel(x, token_table, pos_table):
    B, L = x.shape
    V, E = token_table.shape
    k = _build(B, L, V, E)
    w = k(
        x.T.astype(jnp.int32),
        token_table.T,
        pos_table.T.reshape(E * L),
    )
    return w.transpose(2, 0, 1)


# speedup vs baseline: 1.0289x; 1.0289x over previous
import functools

import jax
import jax.numpy as jnp
from jax import lax
from jax.experimental import pallas as pl
from jax.experimental.pallas import tpu as pltpu
from jax.experimental.pallas import tpu_sc as plsc

NC = 2
NS = 16
NW = NC * NS
LANES = 16
LB = 2


@functools.lru_cache(maxsize=None)
def _build(B, L, V, E):
    assert E % NW == 0 and L % LB == 0 and B % LANES == 0
    e_per_w = E // NW
    n_blk = L // LB

    mesh = plsc.VectorSubcoreMesh(
        core_axis_name="c", subcore_axis_name="s", num_cores=NC, num_subcores=NS
    )

    def body(xt_hbm, tokt_hbm, posf_hbm, w_hbm,
             col_v, pos_v, xb0, xb1, ob0, ob1, cs, xs0, xs1, os0, os1):
        xbufs = (xb0, xb1)
        obufs = (ob0, ob1)
        xsems = (xs0, xs1)
        osems = (os0, os1)
        wid = lax.axis_index("s") * NC + lax.axis_index("c")
        e0 = wid * e_per_w

        pltpu.sync_copy(posf_hbm, pos_v)

        def xload(blk, b):
            return pltpu.make_async_copy(
                xt_hbm.at[pl.ds(blk * LB, LB)], xbufs[b], xsems[b]
            )

        def oscatter(blk, e, b):
            return pltpu.make_async_copy(
                obufs[b], w_hbm.at[pl.ds(blk * LB, LB), e], osems[b]
            )

        def compute_blk(blk, e, b):
            xb, ob = xbufs[b], obufs[b]
            pos_base = e * L
            for lb in range(LB):
                l = blk * LB + lb
                pos = plsc.load_gather(
                    pos_v, [jnp.full((LANES,), pos_base + l, jnp.int32)]
                )

                def per_k(k, _, xb=xb, ob=ob, lb=lb, pos=pos):
                    c = k * LANES
                    v = xb[lb, pl.ds(c, LANES)]
                    val = plsc.load_gather(col_v, [v])
                    ob[lb, pl.ds(c, LANES)] = val + pos
                    return 0

                lax.fori_loop(0, B // LANES, per_k, 0, unroll=8)

        for p in range(e_per_w):
            e = e0 + p
            pltpu.async_copy(tokt_hbm.at[e], col_v, cs).wait()

            xload(0, 0).start()
            xload(1, 1).start()

            def pair_body(t, _, e=e):
                for b in range(2):
                    blk = 2 * t + b
                    xload(blk, b).wait()

                    @pl.when(t > 0)
                    def _():
                        oscatter(blk - 2, e, b).wait()

                    compute_blk(blk, e, b)

                    @pl.when(blk + 2 < n_blk)
                    def _():
                        xload(blk + 2, b).start()

                    oscatter(blk, e, b).start()
                return 0

            lax.fori_loop(0, n_blk // 2, pair_body, 0)

            oscatter(n_blk - 2, e, 0).wait()
            oscatter(n_blk - 1, e, 1).wait()

    return pl.kernel(
        body,
        out_type=jax.ShapeDtypeStruct((L, E, B), jnp.float32),
        mesh=mesh,
        compiler_params=pltpu.CompilerParams(
            use_tc_tiling_on_sc=False, needs_layout_passes=False
        ),
        scratch_types=[
            pltpu.VMEM((V,), jnp.float32),
            pltpu.VMEM((E * L,), jnp.float32),
            pltpu.VMEM((LB, B), jnp.int32),
            pltpu.VMEM((LB, B), jnp.int32),
            pltpu.VMEM((LB, B), jnp.float32),
            pltpu.VMEM((LB, B), jnp.float32),
            pltpu.SemaphoreType.DMA,
            pltpu.SemaphoreType.DMA,
            pltpu.SemaphoreType.DMA,
            pltpu.SemaphoreType.DMA,
            pltpu.SemaphoreType.DMA,
        ],
    )


def kernel(x, token_table, pos_table):
    B, L = x.shape
    V, E = token_table.shape
    k = _build(B, L, V, E)
    w = k(
        x.T.astype(jnp.int32),
        token_table.T,
        pos_table.T.reshape(E * L),
    )
    return w.transpose(2, 0, 1)


# speedup vs baseline: 1.2321x; 1.1975x over previous
import functools

import jax
import jax.numpy as jnp
from jax import lax
from jax.experimental import pallas as pl
from jax.experimental.pallas import tpu as pltpu
from jax.experimental.pallas import tpu_sc as plsc

NC = 2
NS = 16
NW = NC * NS
LANES = 16
LB = 2


@functools.lru_cache(maxsize=None)
def _build(B, L, V, E):
    assert E % NW == 0 and L % LB == 0 and B % LANES == 0
    e_per_w = E // NW
    n_blk = L // LB

    mesh = plsc.VectorSubcoreMesh(
        core_axis_name="c", subcore_axis_name="s", num_cores=NC, num_subcores=NS
    )

    def body(xt_hbm, tokt_hbm, posf_hbm, w_hbm,
             col_v, pos_v, xb0, xb1, ob0, ob1, cs, xs0, xs1, os0, os1):
        xbufs = (xb0, xb1)
        obufs = (ob0, ob1)
        xsems = (xs0, xs1)
        osems = (os0, os1)
        wid = lax.axis_index("s") * NC + lax.axis_index("c")
        e0 = wid * e_per_w

        pltpu.sync_copy(posf_hbm, pos_v)

        def xload(blk, b):
            return pltpu.make_async_copy(
                xt_hbm.at[pl.ds(blk * LB, LB)], xbufs[b], xsems[b]
            )

        def oscatter(blk, e, b):
            return pltpu.make_async_copy(
                obufs[b], w_hbm.at[pl.ds(blk * LB, LB), e], osems[b]
            )

        def compute_blk(blk, e, b):
            xb, ob = xbufs[b], obufs[b]
            pos_base = e * L
            for lb in range(LB):
                l = blk * LB + lb
                pos = plsc.load_gather(
                    pos_v, [jnp.full((LANES,), pos_base + l, jnp.int32)]
                )

                @plsc.parallel_loop(0, B, LANES, unroll=8)
                def per_k(c, xb=xb, ob=ob, lb=lb, pos=pos):
                    v = xb[lb, pl.ds(c, LANES)]
                    val = plsc.load_gather(col_v, [v])
                    ob[lb, pl.ds(c, LANES)] = val + pos

        for p in range(e_per_w):
            e = e0 + p
            pltpu.async_copy(tokt_hbm.at[e], col_v, cs).wait()

            xload(0, 0).start()
            xload(1, 1).start()

            def pair_body(t, _, e=e):
                for b in range(2):
                    blk = 2 * t + b
                    xload(blk, b).wait()

                    @pl.when(t > 0)
                    def _():
                        oscatter(blk - 2, e, b).wait()

                    compute_blk(blk, e, b)

                    @pl.when(blk + 2 < n_blk)
                    def _():
                        xload(blk + 2, b).start()

                    oscatter(blk, e, b).start()
                return 0

            lax.fori_loop(0, n_blk // 2, pair_body, 0)

            oscatter(n_blk - 2, e, 0).wait()
            oscatter(n_blk - 1, e, 1).wait()

    return pl.kernel(
        body,
        out_type=jax.ShapeDtypeStruct((L, E, B), jnp.float32),
        mesh=mesh,
        compiler_params=pltpu.CompilerParams(
            use_tc_tiling_on_sc=False, needs_layout_passes=False
        ),
        scratch_types=[
            pltpu.VMEM((V,), jnp.float32),
            pltpu.VMEM((E * L,), jnp.float32),
            pltpu.VMEM((LB, B), jnp.int32),
            pltpu.VMEM((LB, B), jnp.int32),
            pltpu.VMEM((LB, B), jnp.float32),
            pltpu.VMEM((LB, B), jnp.float32),
            pltpu.SemaphoreType.DMA,
            pltpu.SemaphoreType.DMA,
            pltpu.SemaphoreType.DMA,
            pltpu.SemaphoreType.DMA,
            pltpu.SemaphoreType.DMA,
        ],
    )


def kernel(x, token_table, pos_table):
    B, L = x.shape
    V, E = token_table.shape
    k = _build(B, L, V, E)
    w = k(
        x.T.astype(jnp.int32),
        token_table.T,
        pos_table.T.reshape(E * L),
    )
    return w.transpose(2, 0, 1)


# speedup vs baseline: 1.3809x; 1.1208x over previous
import functools

import jax
import jax.numpy as jnp
from jax import lax
from jax.experimental import pallas as pl
from jax.experimental.pallas import tpu as pltpu
from jax.experimental.pallas import tpu_sc as plsc

NC = 2
NS = 16
NW = NC * NS
LANES = 16
LB = 4


@functools.lru_cache(maxsize=None)
def _build(B, L, V, E):
    assert E % NW == 0 and L % LB == 0 and B % LANES == 0
    e_per_w = E // NW
    n_blk = L // LB

    mesh = plsc.VectorSubcoreMesh(
        core_axis_name="c", subcore_axis_name="s", num_cores=NC, num_subcores=NS
    )

    def body(xt_hbm, tokt_hbm, posf_hbm, w_hbm,
             col_v, pos_v, xb0, xb1, ob0, ob1, cs, xs0, xs1, os0, os1):
        xbufs = (xb0, xb1)
        obufs = (ob0, ob1)
        xsems = (xs0, xs1)
        osems = (os0, os1)
        wid = lax.axis_index("s") * NC + lax.axis_index("c")
        e0 = wid * e_per_w

        pltpu.sync_copy(posf_hbm, pos_v)

        def xload(blk, b):
            return pltpu.make_async_copy(
                xt_hbm.at[pl.ds(blk * LB, LB)], xbufs[b], xsems[b]
            )

        def oscatter(blk, e, b):
            return pltpu.make_async_copy(
                obufs[b], w_hbm.at[pl.ds(blk * LB, LB), e], osems[b]
            )

        def compute_blk(blk, e, b):
            xb, ob = xbufs[b], obufs[b]
            pos_base = e * L
            for lb in range(LB):
                l = blk * LB + lb
                pos = plsc.load_gather(
                    pos_v, [jnp.full((LANES,), pos_base + l, jnp.int32)]
                )

                @plsc.parallel_loop(0, B, LANES, unroll=16)
                def per_k(c, xb=xb, ob=ob, lb=lb, pos=pos):
                    v = xb[lb, pl.ds(c, LANES)]
                    val = plsc.load_gather(col_v, [v])
                    ob[lb, pl.ds(c, LANES)] = val + pos

        for p in range(e_per_w):
            e = e0 + p
            pltpu.async_copy(tokt_hbm.at[e], col_v, cs).wait()

            xload(0, 0).start()
            xload(1, 1).start()

            def pair_body(t, _, e=e):
                for b in range(2):
                    blk = 2 * t + b
                    xload(blk, b).wait()

                    @pl.when(t > 0)
                    def _():
                        oscatter(blk - 2, e, b).wait()

                    compute_blk(blk, e, b)

                    @pl.when(blk + 2 < n_blk)
                    def _():
                        xload(blk + 2, b).start()

                    oscatter(blk, e, b).start()
                return 0

            lax.fori_loop(0, n_blk // 2, pair_body, 0)

            oscatter(n_blk - 2, e, 0).wait()
            oscatter(n_blk - 1, e, 1).wait()

    return pl.kernel(
        body,
        out_type=jax.ShapeDtypeStruct((L, E, B), jnp.float32),
        mesh=mesh,
        compiler_params=pltpu.CompilerParams(
            use_tc_tiling_on_sc=False, needs_layout_passes=False
        ),
        scratch_types=[
            pltpu.VMEM((V,), jnp.float32),
            pltpu.VMEM((E * L,), jnp.float32),
            pltpu.VMEM((LB, B), jnp.int32),
            pltpu.VMEM((LB, B), jnp.int32),
            pltpu.VMEM((LB, B), jnp.float32),
            pltpu.VMEM((LB, B), jnp.float32),
            pltpu.SemaphoreType.DMA,
            pltpu.SemaphoreType.DMA,
            pltpu.SemaphoreType.DMA,
            pltpu.SemaphoreType.DMA,
            pltpu.SemaphoreType.DMA,
        ],
    )


def kernel(x, token_table, pos_table):
    B, L = x.shape
    V, E = token_table.shape
    k = _build(B, L, V, E)
    w = k(
        x.T.astype(jnp.int32),
        token_table.T,
        pos_table.T.reshape(E * L),
    )
    return w.transpose(2, 0, 1)
